# Initial kernel scaffold; baseline (speedup 1.0000x reference)
import jax
import jax.numpy as jnp
from jax.experimental import pallas as pl


def kernel(x, edge_index, edge_attr, W_node, b_node, W_e1, b_e1, W_e2, b_e2, W_root, b_conv, W_ih, W_hh, b_ih, b_hh):
    raise NotImplementedError("write your pallas kernel here")


# speedup vs baseline: 2.5518x; 2.5518x over previous
import functools

import jax
import jax.numpy as jnp
from jax import lax
from jax.experimental import pallas as pl
from jax.experimental.pallas import tpu as pltpu
from jax.experimental.pallas import tpu_sc as plsc

N = 10000
E = 160000
D_IN = 128
D_EDGE = 16
H = 32
EH = 128
STEPS = 3

NC = 2
NS = 16
NW = NC * NS
CH = 128
NCHUNK = E // CH
CPW = -(-NCHUNK // NW)
ROWS_PER_SUB = N // NS


def _node_init_body(x_ref, w_ref, b_ref, o_ref):
    acc = jnp.dot(x_ref[...], w_ref[...], preferred_element_type=jnp.float32)
    o_ref[...] = jax.nn.relu(acc + b_ref[...])


def _node_init(x, WnT, b_row):
    R = 2000
    return pl.pallas_call(
        _node_init_body,
        grid=(N // R,),
        in_specs=[
            pl.BlockSpec((R, D_IN), lambda i: (i, 0)),
            pl.BlockSpec((D_IN, H), lambda i: (0, 0)),
            pl.BlockSpec((1, H), lambda i: (0, 0)),
        ],
        out_specs=pl.BlockSpec((R, H), lambda i: (i, 0)),
        out_shape=jax.ShapeDtypeStruct((N, H), jnp.float32),
    )(x, WnT, b_row)


def _hid_body(w_ref, ea_ref, b_ref, o_ref):
    h = jnp.dot(w_ref[...], ea_ref[...], preferred_element_type=jnp.float32)
    o_ref[...] = jax.nn.relu(h + b_ref[...]).astype(jnp.bfloat16)


def _hid(W_e1, eaT, b_col):
    B = 1280
    return pl.pallas_call(
        _hid_body,
        grid=(E // B,),
        in_specs=[
            pl.BlockSpec((EH, D_EDGE), lambda i: (0, 0)),
            pl.BlockSpec((D_EDGE, B), lambda i: (0, i)),
            pl.BlockSpec((EH, 1), lambda i: (0, 0)),
        ],
        out_specs=pl.BlockSpec((EH, B), lambda i: (0, i)),
        out_shape=jax.ShapeDtypeStruct((EH, E), jnp.bfloat16),
    )(W_e1, eaT, b_col)


def _msg_body(w2_ref, b2_ref, hidT_ref, xj_ref, o_ref):
    ewT = jnp.dot(w2_ref[...], hidT_ref[...], preferred_element_type=jnp.float32)
    ewT = ewT + b2_ref[...]
    ew3 = ewT.reshape(H, H, ewT.shape[-1])
    xjT = xj_ref[...].T
    msgT = jnp.sum(ew3 * xjT[:, None, :], axis=0)
    o_ref[...] = msgT.T


def _msg(W_e2b, b2_col, hidT, xj):
    B = 640
    return pl.pallas_call(
        _msg_body,
        grid=(E // B,),
        in_specs=[
            pl.BlockSpec((H * H, EH), lambda i: (0, 0)),
            pl.BlockSpec((H * H, 1), lambda i: (0, 0)),
            pl.BlockSpec((EH, B), lambda i: (0, i)),
            pl.BlockSpec((B, H), lambda i: (i, 0)),
        ],
        out_specs=pl.BlockSpec((B, H), lambda i: (i, 0)),
        out_shape=jax.ShapeDtypeStruct((E, H), jnp.float32),
    )(W_e2b, b2_col, hidT, xj)


def _update_body(a2_ref, node_ref,
                 wr_ref, wir_ref, wiz_ref, win_ref, whr_ref, whz_ref, whn_ref,
                 bc_ref, bir_ref, biz_ref, bin_ref, bhr_ref, bhz_ref, bhn_ref,
                 o_ref):
    node = node_ref[...]
    aggr = a2_ref[0] + a2_ref[1]
    conv = aggr + jnp.dot(node, wr_ref[...], preferred_element_type=jnp.float32)
    m = jax.nn.relu(conv + bc_ref[...])
    i_r = jnp.dot(m, wir_ref[...], preferred_element_type=jnp.float32) + bir_ref[...]
    i_z = jnp.dot(m, wiz_ref[...], preferred_element_type=jnp.float32) + biz_ref[...]
    i_n = jnp.dot(m, win_ref[...], preferred_element_type=jnp.float32) + bin_ref[...]
    h_r = jnp.dot(node, whr_ref[...], preferred_element_type=jnp.float32) + bhr_ref[...]
    h_z = jnp.dot(node, whz_ref[...], preferred_element_type=jnp.float32) + bhz_ref[...]
    h_n = jnp.dot(node, whn_ref[...], preferred_element_type=jnp.float32) + bhn_ref[...]
    r = jax.nn.sigmoid(i_r + h_r)
    z = jax.nn.sigmoid(i_z + h_z)
    ng = jnp.tanh(i_n + r * h_n)
    o_ref[...] = (1.0 - z) * ng + z * node


def _update(a2, node, mats, biases):
    R = 2000
    w_spec = pl.BlockSpec((H, H), lambda i: (0, 0))
    b_spec = pl.BlockSpec((1, H), lambda i: (0, 0))
    return pl.pallas_call(
        _update_body,
        grid=(N // R,),
        in_specs=[
            pl.BlockSpec((NC, R, H), lambda i: (0, i, 0)),
            pl.BlockSpec((R, H), lambda i: (i, 0)),
        ] + [w_spec] * 7 + [b_spec] * 7,
        out_specs=pl.BlockSpec((R, H), lambda i: (i, 0)),
        out_shape=jax.ShapeDtypeStruct((N, H), jnp.float32),
    )(a2, node, *mats, *biases)


_sc_cache = {}


def _sc_kernels():
    if "gather" in _sc_cache:
        return _sc_cache["gather"], _sc_cache["scatter"]

    mesh = plsc.VectorSubcoreMesh(core_axis_name="c", subcore_axis_name="s",
                                  num_cores=NC, num_subcores=NS)
    cp = pltpu.CompilerParams(use_tc_tiling_on_sc=False)

    @functools.partial(
        pl.kernel,
        out_type=jax.ShapeDtypeStruct((E, H), jnp.float32),
        mesh=mesh,
        compiler_params=cp,
        scratch_types=[
            pltpu.VMEM((CH,), jnp.int32),
            pltpu.VMEM((CH, H), jnp.float32),
            pltpu.SemaphoreType.DMA,
        ],
    )
    def gather_k(node_hbm, src_hbm, out_hbm, idx_v, rows_v, sem):
        wid = lax.axis_index("s") * NC + lax.axis_index("c")

        @pl.loop(0, CPW)
        def _chunk(i):
            c = wid * CPW + i

            @pl.when(c < NCHUNK)
            def _():
                off = c * CH
                pltpu.sync_copy(src_hbm.at[pl.ds(off, CH)], idx_v)
                pltpu.async_copy(node_hbm.at[idx_v], rows_v, sem).wait()
                pltpu.sync_copy(rows_v, out_hbm.at[pl.ds(off, CH)])

    @functools.partial(
        pl.kernel,
        out_type=jax.ShapeDtypeStruct((NC, N, H), jnp.float32),
        mesh=mesh,
        compiler_params=cp,
        scratch_types=[
            pltpu.VMEM((CH,), jnp.int32),
            pltpu.VMEM((CH, H), jnp.float32),
            pltpu.VMEM_SHARED((N, H), jnp.float32),
            pltpu.SemaphoreType.DMA,
        ],
    )
    def scatter_k(msg_hbm, dst_hbm, zero_hbm, out_hbm, idx_v, row_v, acc_sh, sem):
        cid = lax.axis_index("c")
        sid = lax.axis_index("s")
        wid = sid * NC + cid
        r0 = sid * ROWS_PER_SUB
        pltpu.sync_copy(zero_hbm.at[pl.ds(r0, ROWS_PER_SUB)],
                        acc_sh.at[pl.ds(r0, ROWS_PER_SUB)])
        plsc.subcore_barrier()

        @pl.loop(0, CPW)
        def _chunk(i):
            c = wid * CPW + i

            @pl.when(c < NCHUNK)
            def _():
                off = c * CH
                pltpu.sync_copy(dst_hbm.at[pl.ds(off, CH)], idx_v)
                pltpu.sync_copy(msg_hbm.at[pl.ds(off, CH)], row_v)
                pltpu.sync_copy(row_v, acc_sh.at[idx_v], add=True)

        plsc.subcore_barrier()
        pltpu.sync_copy(acc_sh.at[pl.ds(r0, ROWS_PER_SUB)],
                        out_hbm.at[cid].at[pl.ds(r0, ROWS_PER_SUB)])

    _sc_cache["gather"] = gather_k
    _sc_cache["scatter"] = scatter_k
    return gather_k, scatter_k


def _sc_gather(node, src):
    gather_k, _ = _sc_kernels()
    return gather_k(node, src)


def _sc_scatter_add(msg, dst, zeros_nh):
    _, scatter_k = _sc_kernels()
    return scatter_k(msg, dst, zeros_nh)


def kernel(x, edge_index, edge_attr, W_node, b_node, W_e1, b_e1, W_e2, b_e2,
           W_root, b_conv, W_ih, W_hh, b_ih, b_hh):
    src = edge_index[0]
    dst = edge_index[1]

    WnT = W_node.T
    eaT = edge_attr.T
    b_e1c = b_e1.reshape(EH, 1)
    W_e2b = W_e2.astype(jnp.bfloat16)
    b_e2c = b_e2.reshape(H * H, 1)
    zeros_nh = jnp.zeros((N, H), jnp.float32)

    mats = (
        W_root.T,
        W_ih[0:H].T, W_ih[H:2 * H].T, W_ih[2 * H:3 * H].T,
        W_hh[0:H].T, W_hh[H:2 * H].T, W_hh[2 * H:3 * H].T,
    )
    biases = (
        b_conv.reshape(1, H),
        b_ih[0:H].reshape(1, H), b_ih[H:2 * H].reshape(1, H),
        b_ih[2 * H:3 * H].reshape(1, H),
        b_hh[0:H].reshape(1, H), b_hh[H:2 * H].reshape(1, H),
        b_hh[2 * H:3 * H].reshape(1, H),
    )

    node = _node_init(x, WnT, b_node.reshape(1, H))
    hidT = _hid(W_e1, eaT, b_e1c)

    for _ in range(STEPS):
        xj = _sc_gather(node, src)
        msg = _msg(W_e2b, b_e2c, hidT, xj)
        a2 = _sc_scatter_add(msg, dst, zeros_nh)
        node = _update(a2, node, mats, biases)

    return node


# speedup vs baseline: 3.0460x; 1.1937x over previous
import functools

import jax
import jax.numpy as jnp
from jax import lax
from jax.experimental import pallas as pl
from jax.experimental.pallas import tpu as pltpu
from jax.experimental.pallas import tpu_sc as plsc

N = 10000
E = 160000
D_IN = 128
D_EDGE = 16
H = 32
EH = 128
STEPS = 3

NC = 2
NS = 16
NW = NC * NS
IB = 125
NSTR = 8
SUP = IB * NSTR
NSUP = E // SUP
SUPW = NSUP // NW
ROWS_PER_SUB = N // NS


def _node_init_body(x_ref, w_ref, b_ref, o_ref):
    acc = jnp.dot(x_ref[...], w_ref[...], preferred_element_type=jnp.float32)
    o_ref[...] = jax.nn.relu(acc + b_ref[...])


def _node_init(x, WnT, b_row):
    R = 2000
    return pl.pallas_call(
        _node_init_body,
        grid=(N // R,),
        in_specs=[
            pl.BlockSpec((R, D_IN), lambda i: (i, 0)),
            pl.BlockSpec((D_IN, H), lambda i: (0, 0)),
            pl.BlockSpec((1, H), lambda i: (0, 0)),
        ],
        out_specs=pl.BlockSpec((R, H), lambda i: (i, 0)),
        out_shape=jax.ShapeDtypeStruct((N, H), jnp.float32),
    )(x, WnT, b_row)


def _hid_body(w_ref, ea_ref, b_ref, o_ref):
    h = jnp.dot(w_ref[...], ea_ref[...], preferred_element_type=jnp.float32)
    o_ref[...] = jax.nn.relu(h + b_ref[...]).astype(jnp.bfloat16)


def _hid(W_e1, eaT, b_col):
    B = 1280
    return pl.pallas_call(
        _hid_body,
        grid=(E // B,),
        in_specs=[
            pl.BlockSpec((EH, D_EDGE), lambda i: (0, 0)),
            pl.BlockSpec((D_EDGE, B), lambda i: (0, i)),
            pl.BlockSpec((EH, 1), lambda i: (0, 0)),
        ],
        out_specs=pl.BlockSpec((EH, B), lambda i: (0, i)),
        out_shape=jax.ShapeDtypeStruct((EH, E), jnp.bfloat16),
    )(W_e1, eaT, b_col)


def _msg_body(w2_ref, b2t_ref, eye_ref, hidT_ref, xj_ref, o_ref):
    ewT = jnp.dot(w2_ref[...], hidT_ref[...], preferred_element_type=jnp.float32)
    ew3 = ewT.reshape(H, H, ewT.shape[-1])
    xjT = lax.dot_general(eye_ref[...], xj_ref[...], (((1,), (1,)), ((), ())),
                          preferred_element_type=jnp.float32)
    msgT = jnp.sum(ew3 * xjT[:, None, :], axis=0)
    msgT = msgT + jnp.dot(b2t_ref[...], xjT, preferred_element_type=jnp.float32)
    o_ref[...] = lax.dot_general(msgT, eye_ref[...], (((0,), (0,)), ((), ())),
                                 preferred_element_type=jnp.float32)


def _msg(W_e2b, b2t, eye32, hidT, xj):
    B = 1280
    return pl.pallas_call(
        _msg_body,
        grid=(E // B,),
        in_specs=[
            pl.BlockSpec((H * H, EH), lambda i: (0, 0)),
            pl.BlockSpec((H, H), lambda i: (0, 0)),
            pl.BlockSpec((H, H), lambda i: (0, 0)),
            pl.BlockSpec((EH, B), lambda i: (0, i)),
            pl.BlockSpec((B, H), lambda i: (i, 0)),
        ],
        out_specs=pl.BlockSpec((B, H), lambda i: (i, 0)),
        out_shape=jax.ShapeDtypeStruct((E, H), jnp.float32),
    )(W_e2b, b2t, eye32, hidT, xj)


def _update_body(a2_ref, node_ref,
                 wr_ref, wir_ref, wiz_ref, win_ref, whr_ref, whz_ref, whn_ref,
                 bc_ref, bir_ref, biz_ref, bin_ref, bhr_ref, bhz_ref, bhn_ref,
                 o_ref):
    node = node_ref[...]
    aggr = a2_ref[0] + a2_ref[1]
    conv = aggr + jnp.dot(node, wr_ref[...], preferred_element_type=jnp.float32)
    m = jax.nn.relu(conv + bc_ref[...])
    i_r = jnp.dot(m, wir_ref[...], preferred_element_type=jnp.float32) + bir_ref[...]
    i_z = jnp.dot(m, wiz_ref[...], preferred_element_type=jnp.float32) + biz_ref[...]
    i_n = jnp.dot(m, win_ref[...], preferred_element_type=jnp.float32) + bin_ref[...]
    h_r = jnp.dot(node, whr_ref[...], preferred_element_type=jnp.float32) + bhr_ref[...]
    h_z = jnp.dot(node, whz_ref[...], preferred_element_type=jnp.float32) + bhz_ref[...]
    h_n = jnp.dot(node, whn_ref[...], preferred_element_type=jnp.float32) + bhn_ref[...]
    r = jax.nn.sigmoid(i_r + h_r)
    z = jax.nn.sigmoid(i_z + h_z)
    ng = jnp.tanh(i_n + r * h_n)
    o_ref[...] = (1.0 - z) * ng + z * node


def _update(a2, node, mats, biases):
    R = 2000
    w_spec = pl.BlockSpec((H, H), lambda i: (0, 0))
    b_spec = pl.BlockSpec((1, H), lambda i: (0, 0))
    return pl.pallas_call(
        _update_body,
        grid=(N // R,),
        in_specs=[
            pl.BlockSpec((NC, R, H), lambda i: (0, i, 0)),
            pl.BlockSpec((R, H), lambda i: (i, 0)),
        ] + [w_spec] * 7 + [b_spec] * 7,
        out_specs=pl.BlockSpec((R, H), lambda i: (i, 0)),
        out_shape=jax.ShapeDtypeStruct((N, H), jnp.float32),
    )(a2, node, *mats, *biases)


_sc_cache = {}


def _sc_kernels():
    if "gather" in _sc_cache:
        return _sc_cache["gather"], _sc_cache["scatter"]

    mesh = plsc.VectorSubcoreMesh(core_axis_name="c", subcore_axis_name="s",
                                  num_cores=NC, num_subcores=NS)
    cp = pltpu.CompilerParams(use_tc_tiling_on_sc=False)

    @functools.partial(
        pl.kernel,
        out_type=jax.ShapeDtypeStruct((E, H), jnp.float32),
        mesh=mesh,
        compiler_params=cp,
        scratch_types=[
            pltpu.VMEM((NSTR, IB), jnp.int32),
            pltpu.VMEM((SUP, H), jnp.float32),
            pltpu.SemaphoreType.DMA,
        ],
    )
    def gather_k(node_hbm, src2_hbm, out_hbm, idx_v, rows_v, sem):
        wid = lax.axis_index("s") * NC + lax.axis_index("c")

        @pl.loop(0, SUPW)
        def _sup(i):
            s = wid * SUPW + i
            pltpu.sync_copy(src2_hbm.at[pl.ds(s * NSTR, NSTR)], idx_v)
            cps = [pltpu.async_copy(node_hbm.at[idx_v.at[j]],
                                    rows_v.at[pl.ds(j * IB, IB)], sem)
                   for j in range(NSTR)]
            for cp_ in cps:
                cp_.wait()
            pltpu.sync_copy(rows_v, out_hbm.at[pl.ds(s * SUP, SUP)])

    @functools.partial(
        pl.kernel,
        out_type=jax.ShapeDtypeStruct((NC, N, H), jnp.float32),
        mesh=mesh,
        compiler_params=cp,
        scratch_types=[
            pltpu.VMEM((NSTR, IB), jnp.int32),
            pltpu.VMEM((SUP, H), jnp.float32),
            pltpu.VMEM_SHARED((N, H), jnp.float32),
            pltpu.SemaphoreType.DMA,
        ],
    )
    def scatter_k(msg_hbm, dst2_hbm, zero_hbm, out_hbm, idx_v, row_v, acc_sh, sem):
        cid = lax.axis_index("c")
        sid = lax.axis_index("s")
        wid = sid * NC + cid
        r0 = sid * ROWS_PER_SUB
        pltpu.sync_copy(zero_hbm.at[pl.ds(r0, ROWS_PER_SUB)],
                        acc_sh.at[pl.ds(r0, ROWS_PER_SUB)])
        plsc.subcore_barrier()

        @pl.loop(0, SUPW)
        def _sup(i):
            s = wid * SUPW + i
            pltpu.sync_copy(dst2_hbm.at[pl.ds(s * NSTR, NSTR)], idx_v)
            pltpu.sync_copy(msg_hbm.at[pl.ds(s * SUP, SUP)], row_v)
            cps = [pltpu.async_copy(row_v.at[pl.ds(j * IB, IB)],
                                    acc_sh.at[idx_v.at[j]], sem, add=True)
                   for j in range(NSTR)]
            for cp_ in cps:
                cp_.wait()

        plsc.subcore_barrier()
        pltpu.sync_copy(acc_sh.at[pl.ds(r0, ROWS_PER_SUB)],
                        out_hbm.at[cid].at[pl.ds(r0, ROWS_PER_SUB)])

    _sc_cache["gather"] = gather_k
    _sc_cache["scatter"] = scatter_k
    return gather_k, scatter_k


def _sc_gather(node, src2):
    gather_k, _ = _sc_kernels()
    return gather_k(node, src2)


def _sc_scatter_add(msg, dst2, zeros_nh):
    _, scatter_k = _sc_kernels()
    return scatter_k(msg, dst2, zeros_nh)


def kernel(x, edge_index, edge_attr, W_node, b_node, W_e1, b_e1, W_e2, b_e2,
           W_root, b_conv, W_ih, W_hh, b_ih, b_hh):
    src2 = edge_index[0].reshape(NSUP * NSTR, IB)
    dst2 = edge_index[1].reshape(NSUP * NSTR, IB)

    WnT = W_node.T
    eaT = edge_attr.T
    b_e1c = b_e1.reshape(EH, 1)
    W_e2b = W_e2.astype(jnp.bfloat16)
    b2t = b_e2.reshape(H, H).T
    eye32 = jnp.eye(H, dtype=jnp.float32)
    zeros_nh = jnp.zeros((N, H), jnp.float32)

    mats = (
        W_root.T,
        W_ih[0:H].T, W_ih[H:2 * H].T, W_ih[2 * H:3 * H].T,
        W_hh[0:H].T, W_hh[H:2 * H].T, W_hh[2 * H:3 * H].T,
    )
    biases = (
        b_conv.reshape(1, H),
        b_ih[0:H].reshape(1, H), b_ih[H:2 * H].reshape(1, H),
        b_ih[2 * H:3 * H].reshape(1, H),
        b_hh[0:H].reshape(1, H), b_hh[H:2 * H].reshape(1, H),
        b_hh[2 * H:3 * H].reshape(1, H),
    )

    node = _node_init(x, WnT, b_node.reshape(1, H))
    hidT = _hid(W_e1, eaT, b_e1c)

    for _ in range(STEPS):
        xj = _sc_gather(node, src2)
        msg = _msg(W_e2b, b2t, eye32, hidT, xj)
        a2 = _sc_scatter_add(msg, dst2, zeros_nh)
        node = _update(a2, node, mats, biases)

    return node
